# Initial kernel scaffold; baseline (speedup 1.0000x reference)
import jax
import jax.numpy as jnp
from jax.experimental import pallas as pl


def kernel(wav_input, W0, b0, W1, b1, W2, b2, W3, b3, W4, b4, codebook):
    raise NotImplementedError("write your pallas kernel here")


# speedup vs baseline: 1.0009x; 1.0009x over previous
import jax
import jax.numpy as jnp
from jax.experimental import pallas as pl

_T5 = 297
_TILE = 152
_WROWS = 312


def _dot(a, b):
    return jax.lax.dot_general(
        a.astype(jnp.bfloat16), b.astype(jnp.bfloat16),
        (((1,), (0,)), ((), ())), preferred_element_type=jnp.float32)


def _phase_conv(xs, taps, bias, s_out):
    p_in = len(xs)
    outs = []
    for q in range(p_in // 2):
        acc = None
        for k in range(taps.shape[0]):
            u = 2 * q + k
            src = xs[u % p_in]
            j = u // p_in
            term = _dot(jax.lax.slice(src, (j, 0), (j + s_out, src.shape[1])),
                        taps[k])
            acc = term if acc is None else acc + term
        outs.append(jnp.maximum(acc + bias, 0.0))
    return outs


def _body(wav_ref, w0_ref, b0_ref, w1_ref, b1_ref, w2_ref, b2_ref,
          w3_ref, b3_ref, w4_ref, b4_ref, cbt_ref, i0_ref, i1_ref):
    t = pl.program_id(1)
    wt = wav_ref[0, pl.ds(_TILE * t, 158), :]
    b0 = b0_ref[...]
    w0 = w0_ref[...]
    x = []
    for p in range(16):
        if p < 15:
            v = _dot(wt[:157, 5 * p:5 * p + 10], w0)
        else:
            pat = jnp.concatenate([wt[:157, 75:80], wt[1:158, 0:5]], axis=1)
            v = _dot(pat, w0)
        x.append(jnp.maximum(v + b0, 0.0))
    x = _phase_conv(x, w1_ref[...], b1_ref[...], 156)
    x = _phase_conv(x, w2_ref[...], b2_ref[...], 155)
    x = _phase_conv(x, w3_ref[...], b3_ref[...], 154)
    x = _phase_conv(x, w4_ref[...], b4_ref[...], _TILE + 1)
    z = x[0][:_TILE]
    for g, out_ref in ((0, i0_ref), (1, i1_ref)):
        zg = z[:, g * 256:(g + 1) * 256]
        cbt = cbt_ref[g]
        zn = jnp.sum(zg * zg, axis=1, keepdims=True)
        en = jnp.sum(cbt * cbt, axis=0, keepdims=True)
        d = (zn - 2.0 * _dot(zg, cbt)) + en
        dmin = jnp.min(d, axis=1, keepdims=True)
        ii = jax.lax.broadcasted_iota(jnp.int32, d.shape, 1)
        idx = jnp.min(jnp.where(d <= dmin, ii, 320), axis=1)
        out_ref[0, 0, 0, :] = idx


def kernel(wav_input, W0, b0, W1, b1, W2, b2, W3, b3, W4, b4, codebook):
    B = wav_input.shape[0]
    wav = jnp.pad(wav_input, ((0, 0), (0, _WROWS * 80 - 24000)))
    wav = wav.reshape(B, _WROWS, 80)
    w0 = W0.reshape(512, 10).T
    w1 = jnp.transpose(W1, (2, 1, 0))
    w2 = jnp.transpose(W2, (2, 1, 0))
    w3 = jnp.transpose(W3, (2, 1, 0))
    w4 = jnp.transpose(W4, (2, 1, 0))
    cbt = jnp.transpose(codebook, (1, 2, 0))
    biases = [b.reshape(1, 512) for b in (b0, b1, b2, b3, b4)]

    full = lambda a: pl.BlockSpec(a.shape, lambda b, t: (0,) * a.ndim)
    in_specs = [pl.BlockSpec((1, _WROWS, 80), lambda b, t: (b, 0, 0))]
    ops = [w0, biases[0], w1, biases[1], w2, biases[2],
           w3, biases[3], w4, biases[4], cbt]
    in_specs += [full(a) for a in ops]

    i0, i1 = pl.pallas_call(
        _body,
        grid=(B, 2),
        in_specs=in_specs,
        out_specs=[pl.BlockSpec((1, 1, 1, _TILE),
                                lambda b, t: (b, t, 0, 0))] * 2,
        out_shape=[jax.ShapeDtypeStruct((B, 2, 1, _TILE), jnp.int32)] * 2,
    )(wav, *ops)

    i0 = i0.reshape(B, 2 * _TILE)[:, :_T5]
    i1 = i1.reshape(B, 2 * _TILE)[:, :_T5]
    idx = jnp.stack([i0, i1], axis=-1)
    return idx.reshape(B, _T5 * 2)
